# Initial kernel scaffold; baseline (speedup 1.0000x reference)
import jax
import jax.numpy as jnp
from jax.experimental import pallas as pl


def kernel(queries, memory_bank):
    raise NotImplementedError("write your pallas kernel here")


# speedup vs baseline: 11.8806x; 11.8806x over previous
import numpy as np
import jax
import jax.numpy as jnp
from jax.experimental import pallas as pl
from jax.experimental.pallas import tpu as pltpu

_PATCH = 32
_OUT = 448
_SIGMA = 4.0
_TOPK = 10


def _resize_mat(in_size: int, out_size: int) -> np.ndarray:
    scale = out_size / in_size
    sample_f = (np.arange(out_size, dtype=np.float64) + 0.5) / scale - 0.5
    x = np.abs(sample_f[None, :] - np.arange(in_size, dtype=np.float64)[:, None])
    w = np.maximum(0.0, 1.0 - x)
    w = w / w.sum(axis=0, keepdims=True)
    return w.T


def _gauss_mat(n: int, sigma: float) -> np.ndarray:
    radius = int(4.0 * sigma)
    x = np.arange(-radius, radius + 1, dtype=np.float64)
    k = np.exp(-0.5 * (x / sigma) ** 2)
    k = k / k.sum()
    pad = np.pad(np.eye(n), ((radius, radius), (0, 0)), mode="reflect")
    g = np.zeros((n, n))
    for t in range(2 * radius + 1):
        g += k[t] * pad[t : t + n, :]
    return g


_M_POST = jnp.asarray(
    (_gauss_mat(_OUT, _SIGMA) @ _resize_mat(_PATCH, _OUT)).astype(np.float32)
)


def _nn_body(kb: int, nsteps: int, q_ref, m_ref, nn_ref, qbf_ref, acc_ref):
    i = pl.program_id(0)

    @pl.when(i == 0)
    def _init():
        qbf_ref[:] = q_ref[:].astype(jnp.bfloat16)
        acc_ref[:] = jnp.full(acc_ref.shape, -jnp.inf, jnp.float32)

    mb = m_ref[:]
    ss = jnp.sum(mb * mb, axis=1, keepdims=True)
    inv = 1.0 / (jnp.sqrt(ss) + 1e-8)
    mbf = (mb * inv).astype(jnp.bfloat16)
    sim = jax.lax.dot_general(
        qbf_ref[:], mbf, (((1,), (1,)), ((), ())),
        preferred_element_type=jnp.float32,
    )
    acc_ref[:] = jnp.maximum(acc_ref[:], jnp.max(sim, axis=1))

    @pl.when(i == nsteps - 1)
    def _fin():
        qf = q_ref[:]
        qss = jnp.sum(qf * qf, axis=1)
        rq = 1.0 / (jnp.sqrt(qss) + 1e-8)
        nn_ref[:] = 1.0 - rq * acc_ref[:]


def _post_body(nn_ref, m_ref, score_ref, amap_ref):
    g = nn_ref[:]
    flat = (
        jax.lax.broadcasted_iota(jnp.int32, g.shape, 0) * g.shape[1]
        + jax.lax.broadcasted_iota(jnp.int32, g.shape, 1)
    )
    v = g
    acc = jnp.float32(0.0)
    for _ in range(_TOPK):
        mx = jnp.max(v)
        acc = acc + mx
        imin = jnp.min(jnp.where(v == mx, flat, jnp.int32(1 << 20)))
        v = jnp.where(flat == imin, -jnp.inf, v)
    score_ref[:] = jnp.reshape(acc / _TOPK, (1, 1))

    m = m_ref[:]
    t = jax.lax.dot_general(
        m, g, (((1,), (0,)), ((), ())), preferred_element_type=jnp.float32
    )
    amap_ref[:] = jax.lax.dot_general(
        t, m, (((1,), (1,)), ((), ())), preferred_element_type=jnp.float32
    )


def kernel(queries, memory_bank):
    q, d = queries.shape
    k, _ = memory_bank.shape
    kb = min(2048, k)
    nsteps = k // kb

    nn = pl.pallas_call(
        lambda *a: _nn_body(kb, nsteps, *a),
        grid=(nsteps,),
        in_specs=[
            pl.BlockSpec((q, d), lambda i: (0, 0)),
            pl.BlockSpec((kb, d), lambda i: (i, 0)),
        ],
        out_specs=pl.BlockSpec((q,), lambda i: (0,)),
        out_shape=jax.ShapeDtypeStruct((q,), jnp.float32),
        scratch_shapes=[
            pltpu.VMEM((q, d), jnp.bfloat16),
            pltpu.VMEM((q,), jnp.float32),
        ],
    )(queries, memory_bank)

    grid32 = nn.reshape(_PATCH, _PATCH)
    score, amap = pl.pallas_call(
        _post_body,
        out_shape=(
            jax.ShapeDtypeStruct((1, 1), jnp.float32),
            jax.ShapeDtypeStruct((_OUT, _OUT), jnp.float32),
        ),
    )(grid32, _M_POST)
    return score.reshape(()), amap


# speedup vs baseline: 12.6834x; 1.0676x over previous
import numpy as np
import jax
import jax.numpy as jnp
from jax.experimental import pallas as pl
from jax.experimental.pallas import tpu as pltpu

_PATCH = 32
_OUT = 448
_SIGMA = 4.0
_TOPK = 10


def _resize_mat(in_size: int, out_size: int) -> np.ndarray:
    scale = out_size / in_size
    sample_f = (np.arange(out_size, dtype=np.float64) + 0.5) / scale - 0.5
    x = np.abs(sample_f[None, :] - np.arange(in_size, dtype=np.float64)[:, None])
    w = np.maximum(0.0, 1.0 - x)
    w = w / w.sum(axis=0, keepdims=True)
    return w.T


def _gauss_mat(n: int, sigma: float) -> np.ndarray:
    radius = int(4.0 * sigma)
    x = np.arange(-radius, radius + 1, dtype=np.float64)
    k = np.exp(-0.5 * (x / sigma) ** 2)
    k = k / k.sum()
    pad = np.pad(np.eye(n), ((radius, radius), (0, 0)), mode="reflect")
    g = np.zeros((n, n))
    for t in range(2 * radius + 1):
        g += k[t] * pad[t : t + n, :]
    return g


_M_POST = (_gauss_mat(_OUT, _SIGMA) @ _resize_mat(_PATCH, _OUT)).astype(
    np.float32
)


def _nn_body(kb: int, nsteps: int, q_ref, m_ref, nn_ref, qbf_ref, acc_ref):
    i = pl.program_id(0)

    @pl.when(i == 0)
    def _init():
        qbf_ref[:] = q_ref[:].astype(jnp.bfloat16)
        acc_ref[:] = jnp.full(acc_ref.shape, -jnp.inf, jnp.float32)

    mb = m_ref[:]
    ones = jnp.ones((8, mb.shape[1]), jnp.float32)
    ss = jax.lax.dot_general(
        mb * mb, ones, (((1,), (1,)), ((), ())),
        preferred_element_type=jnp.float32,
    )[:, 0:1]
    inv = 1.0 / (jnp.sqrt(ss) + 1e-8)
    mbf = (mb * inv).astype(jnp.bfloat16)
    sim = jax.lax.dot_general(
        mbf, qbf_ref[:], (((1,), (1,)), ((), ())),
        preferred_element_type=jnp.float32,
    )
    acc_ref[:] = jnp.maximum(acc_ref[:], jnp.max(sim, axis=0))

    @pl.when(i == nsteps - 1)
    def _fin():
        qf = q_ref[:]
        qss = jnp.sum(qf * qf, axis=1)
        rq = 1.0 / (jnp.sqrt(qss) + 1e-8)
        nn_ref[:] = 1.0 - rq * acc_ref[:]


def _post_body(nn_ref, m_ref, score_ref, amap_ref):
    g = nn_ref[:]
    flat = (
        jax.lax.broadcasted_iota(jnp.int32, g.shape, 0) * g.shape[1]
        + jax.lax.broadcasted_iota(jnp.int32, g.shape, 1)
    )
    v = g
    acc = jnp.float32(0.0)
    for _ in range(_TOPK):
        mx = jnp.max(v)
        acc = acc + mx
        imin = jnp.min(jnp.where(v == mx, flat, jnp.int32(1 << 20)))
        v = jnp.where(flat == imin, -jnp.inf, v)
    score_ref[:] = jnp.reshape(acc / _TOPK, (1, 1))

    m = m_ref[:]
    t = jax.lax.dot_general(
        m, g, (((1,), (0,)), ((), ())), preferred_element_type=jnp.float32
    )
    amap_ref[:] = jax.lax.dot_general(
        t, m, (((1,), (1,)), ((), ())), preferred_element_type=jnp.float32
    )


def kernel(queries, memory_bank):
    q, d = queries.shape
    k, _ = memory_bank.shape
    kb = min(2048, k)
    nsteps = k // kb

    nn = pl.pallas_call(
        lambda *a: _nn_body(kb, nsteps, *a),
        grid=(nsteps,),
        in_specs=[
            pl.BlockSpec((q, d), lambda i: (0, 0)),
            pl.BlockSpec((kb, d), lambda i: (i, 0)),
        ],
        out_specs=pl.BlockSpec((q,), lambda i: (0,)),
        out_shape=jax.ShapeDtypeStruct((q,), jnp.float32),
        scratch_shapes=[
            pltpu.VMEM((q, d), jnp.bfloat16),
            pltpu.VMEM((q,), jnp.float32),
        ],
    )(queries, memory_bank)

    grid32 = nn.reshape(_PATCH, _PATCH)
    score, amap = pl.pallas_call(
        _post_body,
        out_shape=(
            jax.ShapeDtypeStruct((1, 1), jnp.float32),
            jax.ShapeDtypeStruct((_OUT, _OUT), jnp.float32),
        ),
    )(grid32, _M_POST)
    return score.reshape(()), amap


# speedup vs baseline: 12.7392x; 1.0044x over previous
import numpy as np
import jax
import jax.numpy as jnp
from jax.experimental import pallas as pl
from jax.experimental.pallas import tpu as pltpu

_PATCH = 32
_OUT = 448
_SIGMA = 4.0
_TOPK = 10


def _resize_mat(in_size: int, out_size: int) -> np.ndarray:
    scale = out_size / in_size
    sample_f = (np.arange(out_size, dtype=np.float64) + 0.5) / scale - 0.5
    x = np.abs(sample_f[None, :] - np.arange(in_size, dtype=np.float64)[:, None])
    w = np.maximum(0.0, 1.0 - x)
    w = w / w.sum(axis=0, keepdims=True)
    return w.T


def _gauss_mat(n: int, sigma: float) -> np.ndarray:
    radius = int(4.0 * sigma)
    x = np.arange(-radius, radius + 1, dtype=np.float64)
    k = np.exp(-0.5 * (x / sigma) ** 2)
    k = k / k.sum()
    pad = np.pad(np.eye(n), ((radius, radius), (0, 0)), mode="reflect")
    g = np.zeros((n, n))
    for t in range(2 * radius + 1):
        g += k[t] * pad[t : t + n, :]
    return g


_M_POST = (_gauss_mat(_OUT, _SIGMA) @ _resize_mat(_PATCH, _OUT)).astype(
    np.float32
)


def _nn_body(kb: int, nsteps: int, q_ref, m_ref, nn_ref, qbf_ref, acc_ref):
    i = pl.program_id(0)

    @pl.when(i == 0)
    def _init():
        qbf_ref[:] = q_ref[:].astype(jnp.bfloat16)
        acc_ref[:] = jnp.full(acc_ref.shape, -jnp.inf, jnp.float32)

    mb = m_ref[:].astype(jnp.bfloat16)
    ones = jnp.ones((8, mb.shape[1]), jnp.bfloat16)
    ss = jax.lax.dot_general(
        mb * mb, ones, (((1,), (1,)), ((), ())),
        preferred_element_type=jnp.float32,
    )[:, 0:1]
    inv = (1.0 / (jnp.sqrt(ss) + 1e-8)).astype(jnp.bfloat16)
    mbf = mb * inv
    sim = jax.lax.dot_general(
        mbf, qbf_ref[:], (((1,), (1,)), ((), ())),
        preferred_element_type=jnp.float32,
    )
    acc_ref[:] = jnp.maximum(acc_ref[:], jnp.max(sim, axis=0))

    @pl.when(i == nsteps - 1)
    def _fin():
        qf = q_ref[:]
        qss = jnp.sum(qf * qf, axis=1)
        rq = 1.0 / (jnp.sqrt(qss) + 1e-8)
        nn_ref[:] = 1.0 - rq * acc_ref[:]


def _post_body(nn_ref, m_ref, score_ref, amap_ref):
    g = nn_ref[:]
    flat = (
        jax.lax.broadcasted_iota(jnp.int32, g.shape, 0) * g.shape[1]
        + jax.lax.broadcasted_iota(jnp.int32, g.shape, 1)
    )
    v = g
    acc = jnp.float32(0.0)
    for _ in range(_TOPK):
        mx = jnp.max(v)
        acc = acc + mx
        imin = jnp.min(jnp.where(v == mx, flat, jnp.int32(1 << 20)))
        v = jnp.where(flat == imin, -jnp.inf, v)
    score_ref[:] = jnp.reshape(acc / _TOPK, (1, 1))

    m = m_ref[:]
    t = jax.lax.dot_general(
        m, g, (((1,), (0,)), ((), ())), preferred_element_type=jnp.float32
    )
    amap_ref[:] = jax.lax.dot_general(
        t, m, (((1,), (1,)), ((), ())), preferred_element_type=jnp.float32
    )


def kernel(queries, memory_bank):
    q, d = queries.shape
    k, _ = memory_bank.shape
    kb = min(2048, k)
    nsteps = k // kb

    nn = pl.pallas_call(
        lambda *a: _nn_body(kb, nsteps, *a),
        grid=(nsteps,),
        in_specs=[
            pl.BlockSpec((q, d), lambda i: (0, 0)),
            pl.BlockSpec((kb, d), lambda i: (i, 0)),
        ],
        out_specs=pl.BlockSpec((q,), lambda i: (0,)),
        out_shape=jax.ShapeDtypeStruct((q,), jnp.float32),
        scratch_shapes=[
            pltpu.VMEM((q, d), jnp.bfloat16),
            pltpu.VMEM((q,), jnp.float32),
        ],
    )(queries, memory_bank)

    grid32 = nn.reshape(_PATCH, _PATCH)
    score, amap = pl.pallas_call(
        _post_body,
        out_shape=(
            jax.ShapeDtypeStruct((1, 1), jnp.float32),
            jax.ShapeDtypeStruct((_OUT, _OUT), jnp.float32),
        ),
    )(grid32, _M_POST)
    return score.reshape(()), amap
